# baseline (speedup 1.0000x reference)
import functools

import jax
import jax.numpy as jnp
from jax import lax
from jax.experimental import pallas as pl
from jax.experimental.pallas import tpu as pltpu
from jax.experimental.pallas import tpu_sc as plsc

_B, _N, _D = 64, 128, 64
_EVOCAB_PAD = 128
_G = 32


def _sc_gather(W_pe, pe_flat, W_h, h_flat):
    n_idx = pe_flat.shape[0]
    d = W_pe.shape[1]
    info = plsc.get_sparse_core_info()
    nw = info.num_cores * info.num_subcores
    per_w = n_idx // nw
    mesh = plsc.VectorSubcoreMesh(core_axis_name="c", subcore_axis_name="s")

    @functools.partial(
        pl.kernel,
        mesh=mesh,
        compiler_params=pltpu.CompilerParams(use_tc_tiling_on_sc=False),
        out_type=[
            jax.ShapeDtypeStruct((n_idx, d), jnp.float32),
            jax.ShapeDtypeStruct((n_idx, d), jnp.float32),
        ],
        scratch_types=[
            pltpu.VMEM((per_w,), jnp.int32),
            pltpu.VMEM((per_w, d), jnp.float32),
            pltpu.VMEM((per_w,), jnp.int32),
            pltpu.VMEM((per_w, d), jnp.float32),
            pltpu.SemaphoreType.DMA,
            pltpu.SemaphoreType.DMA,
        ],
    )
    def k(wpe_hbm, pe_hbm, wh_hbm, h_hbm, pe_out, h_out,
          idx_pe, rows_pe, idx_h, rows_h, sem_pe, sem_h):
        wid = lax.axis_index("s") * info.num_cores + lax.axis_index("c")
        base = wid * per_w
        pltpu.sync_copy(pe_hbm.at[pl.ds(base, per_w)], idx_pe)
        pltpu.sync_copy(h_hbm.at[pl.ds(base, per_w)], idx_h)
        cp_pe = pltpu.async_copy(wpe_hbm.at[idx_pe], rows_pe, sem_pe)
        cp_h = pltpu.async_copy(wh_hbm.at[idx_h], rows_h, sem_h)
        cp_pe.wait()
        cp_h.wait()
        pltpu.sync_copy(rows_pe, pe_out.at[pl.ds(base, per_w)])
        pltpu.sync_copy(rows_h, h_out.at[pl.ds(base, per_w)])

    return k(W_pe, pe_flat, W_h, h_flat)


def _tc_body(e_ref, et_ref, pe_ref, hrow_ref, wet_ref, eout_ref, hout_ref):
    pe = pe_ref[0]
    pet = pe.T
    hout_ref[0] = hrow_ref[0].T + pet
    g = pl.program_id(1)
    base = g * _G
    eb_rows = e_ref[0, pl.ds(base, _G), :]
    ebt_rows = et_ref[0, pl.ds(base, _G), :]
    row_ids = lax.broadcasted_iota(jnp.int32, (_G, _N), 0) + base
    col_ids = lax.broadcasted_iota(jnp.int32, (_G, _N), 1)
    esym_g = jnp.where(row_ids > col_ids, ebt_rows, eb_rows)
    petg = pe_ref[0, pl.ds(base, _G), :].T
    wet = wet_ref[...].astype(jnp.bfloat16)
    vocab_col = lax.broadcasted_iota(jnp.int32, (_EVOCAB_PAD, 1), 0)
    for k in range(_G // 2):
        ids2 = jnp.concatenate(
            [esym_g[2 * k:2 * k + 1, :], esym_g[2 * k + 1:2 * k + 2, :]],
            axis=1)
        oht = (vocab_col == ids2).astype(jnp.bfloat16)
        res = lax.dot_general(wet, oht, (((1,), (0,)), ((), ())),
                              preferred_element_type=jnp.float32)
        eout_ref[0, 2 * k] = (res[:, :_N] + pet) + petg[:, 2 * k:2 * k + 1]
        eout_ref[0, 2 * k + 1] = (
            res[:, _N:] + pet) + petg[:, 2 * k + 1:2 * k + 2]


def _tc_call(e, e_t, pe_emb, h_rows, wet_pad):
    return pl.pallas_call(
        _tc_body,
        grid=(_B, _N // _G),
        compiler_params=pltpu.CompilerParams(
            dimension_semantics=("arbitrary", "arbitrary"),
            vmem_limit_bytes=100 * 1024 * 1024,
        ),
        in_specs=[
            pl.BlockSpec((1, _N, _N), lambda b, g: (b, 0, 0)),
            pl.BlockSpec((1, _N, _N), lambda b, g: (b, 0, 0)),
            pl.BlockSpec((1, _N, _D), lambda b, g: (b, 0, 0)),
            pl.BlockSpec((1, _N, _D), lambda b, g: (b, 0, 0)),
            pl.BlockSpec((_D, _EVOCAB_PAD), lambda b, g: (0, 0)),
        ],
        out_specs=[
            pl.BlockSpec((1, _G, _D, _N), lambda b, g: (b, g, 0, 0)),
            pl.BlockSpec((1, _D, _N), lambda b, g: (b, 0, 0)),
        ],
        out_shape=[
            jax.ShapeDtypeStruct((_B, _N, _D, _N), jnp.float32),
            jax.ShapeDtypeStruct((_B, _D, _N), jnp.float32),
        ],
    )(e, e_t, pe_emb, h_rows, wet_pad)


def _tc_transpose_e(e):
    def body(e_ref, et_ref):
        et_ref[0] = e_ref[0].T

    return pl.pallas_call(
        body,
        grid=(_B,),
        in_specs=[pl.BlockSpec((1, _N, _N), lambda b: (b, 0, 0))],
        out_specs=pl.BlockSpec((1, _N, _N), lambda b: (b, 0, 0)),
        out_shape=jax.ShapeDtypeStruct((_B, _N, _N), jnp.int32),
    )(e)


@jax.jit
def kernel(h, e, pe, W_h, W_e, W_pe):
    h = h.astype(jnp.int32)
    e = e.astype(jnp.int32)
    pe = pe.astype(jnp.int32)
    pe_emb_flat, h_rows_flat = _sc_gather(
        W_pe, pe.reshape(_B * _N), W_h, h.reshape(_B * _N))
    pe_emb = pe_emb_flat.reshape(_B, _N, _D)
    h_rows = h_rows_flat.reshape(_B, _N, _D)
    wet_pad = jnp.zeros((_D, _EVOCAB_PAD), jnp.float32).at[:, :W_e.shape[0]].set(W_e.T)
    e_t = _tc_transpose_e(e)
    e_out_t, h_out_t = _tc_call(e, e_t, pe_emb, h_rows, wet_pad)
    e_out = jnp.transpose(e_out_t, (0, 1, 3, 2))
    h_out = jnp.transpose(h_out_t, (0, 2, 1))
    return (h_out, e_out)


# speedup vs baseline: 1.4007x; 1.4007x over previous
import functools

import jax
import jax.numpy as jnp
from jax import lax
from jax.experimental import pallas as pl
from jax.experimental.pallas import tpu as pltpu
from jax.experimental.pallas import tpu_sc as plsc

_B, _N, _D = 64, 128, 64
_EVOCAB_PAD = 128
_G = 64


def _sc_gather(W_pe, pe_flat, W_h, h_flat):
    n_idx = pe_flat.shape[0]
    d = W_pe.shape[1]
    info = plsc.get_sparse_core_info()
    nw = info.num_cores * info.num_subcores
    per_w = n_idx // nw
    mesh = plsc.VectorSubcoreMesh(core_axis_name="c", subcore_axis_name="s")

    @functools.partial(
        pl.kernel,
        mesh=mesh,
        compiler_params=pltpu.CompilerParams(use_tc_tiling_on_sc=False),
        out_type=[
            jax.ShapeDtypeStruct((n_idx, d), jnp.float32),
            jax.ShapeDtypeStruct((n_idx, d), jnp.float32),
        ],
        scratch_types=[
            pltpu.VMEM((per_w,), jnp.int32),
            pltpu.VMEM((per_w, d), jnp.float32),
            pltpu.VMEM((per_w,), jnp.int32),
            pltpu.VMEM((per_w, d), jnp.float32),
            pltpu.SemaphoreType.DMA,
            pltpu.SemaphoreType.DMA,
        ],
    )
    def k(wpe_hbm, pe_hbm, wh_hbm, h_hbm, pe_out, h_out,
          idx_pe, rows_pe, idx_h, rows_h, sem_pe, sem_h):
        wid = lax.axis_index("s") * info.num_cores + lax.axis_index("c")
        base = wid * per_w
        pltpu.sync_copy(pe_hbm.at[pl.ds(base, per_w)], idx_pe)
        pltpu.sync_copy(h_hbm.at[pl.ds(base, per_w)], idx_h)
        cp_pe = pltpu.async_copy(wpe_hbm.at[idx_pe], rows_pe, sem_pe)
        cp_h = pltpu.async_copy(wh_hbm.at[idx_h], rows_h, sem_h)
        cp_pe.wait()
        cp_h.wait()
        pltpu.sync_copy(rows_pe, pe_out.at[pl.ds(base, per_w)])
        pltpu.sync_copy(rows_h, h_out.at[pl.ds(base, per_w)])

    return k(W_pe, pe_flat, W_h, h_flat)


def _tc_body(e_ref, et_ref, pe_ref, hrow_ref, wet_ref, eout_ref, hout_ref):
    pe = pe_ref[0]
    pet = pe.T
    hout_ref[0] = hrow_ref[0].T + pet
    g = pl.program_id(1)
    base = g * _G
    eb_rows = e_ref[0, pl.ds(base, _G), :]
    ebt_rows = et_ref[0, pl.ds(base, _G), :]
    row_ids = lax.broadcasted_iota(jnp.int32, (_G, _N), 0) + base
    col_ids = lax.broadcasted_iota(jnp.int32, (_G, _N), 1)
    esym_g = jnp.where(row_ids > col_ids, ebt_rows, eb_rows)
    petg = pe_ref[0, pl.ds(base, _G), :].T
    wet = wet_ref[...].astype(jnp.bfloat16)
    vocab_col = lax.broadcasted_iota(jnp.int32, (_EVOCAB_PAD, 1), 0)
    for k in range(_G // 2):
        ids2 = jnp.concatenate(
            [esym_g[2 * k:2 * k + 1, :], esym_g[2 * k + 1:2 * k + 2, :]],
            axis=1)
        oht = (vocab_col == ids2).astype(jnp.bfloat16)
        res = lax.dot_general(wet, oht, (((1,), (0,)), ((), ())),
                              preferred_element_type=jnp.float32)
        eout_ref[0, 2 * k] = (res[:, :_N] + pet) + petg[:, 2 * k:2 * k + 1]
        eout_ref[0, 2 * k + 1] = (
            res[:, _N:] + pet) + petg[:, 2 * k + 1:2 * k + 2]


def _tc_call(e, e_t, pe_emb, h_rows, wet_pad):
    return pl.pallas_call(
        _tc_body,
        grid=(_B, _N // _G),
        compiler_params=pltpu.CompilerParams(
            dimension_semantics=("arbitrary", "arbitrary"),
            vmem_limit_bytes=100 * 1024 * 1024,
        ),
        in_specs=[
            pl.BlockSpec((1, _N, _N), lambda b, g: (b, 0, 0)),
            pl.BlockSpec((1, _N, _N), lambda b, g: (b, 0, 0)),
            pl.BlockSpec((1, _N, _D), lambda b, g: (b, 0, 0)),
            pl.BlockSpec((1, _N, _D), lambda b, g: (b, 0, 0)),
            pl.BlockSpec((_D, _EVOCAB_PAD), lambda b, g: (0, 0)),
        ],
        out_specs=[
            pl.BlockSpec((1, _G, _D, _N), lambda b, g: (b, g, 0, 0)),
            pl.BlockSpec((1, _D, _N), lambda b, g: (b, 0, 0)),
        ],
        out_shape=[
            jax.ShapeDtypeStruct((_B, _N, _D, _N), jnp.float32),
            jax.ShapeDtypeStruct((_B, _D, _N), jnp.float32),
        ],
    )(e, e_t, pe_emb, h_rows, wet_pad)


@jax.jit
def kernel(h, e, pe, W_h, W_e, W_pe):
    h = h.astype(jnp.int32)
    e = e.astype(jnp.int32)
    pe = pe.astype(jnp.int32)
    pe_emb_flat, h_rows_flat = _sc_gather(
        W_pe, pe.reshape(_B * _N), W_h, h.reshape(_B * _N))
    pe_emb = pe_emb_flat.reshape(_B, _N, _D)
    h_rows = h_rows_flat.reshape(_B, _N, _D)
    wet_pad = jnp.zeros((_D, _EVOCAB_PAD), jnp.float32).at[:, :W_e.shape[0]].set(W_e.T)
    e_t = jnp.swapaxes(e, 1, 2)
    e_out_t, h_out_t = _tc_call(e, e_t, pe_emb, h_rows, wet_pad)
    e_out = jnp.transpose(e_out_t, (0, 1, 3, 2))
    h_out = jnp.transpose(h_out_t, (0, 2, 1))
    return (h_out, e_out)


# speedup vs baseline: 1.5707x; 1.1214x over previous
import functools

import jax
import jax.numpy as jnp
from jax import lax
from jax.experimental import pallas as pl
from jax.experimental.pallas import tpu as pltpu
from jax.experimental.pallas import tpu_sc as plsc

_B, _N, _D = 64, 128, 64
_EVOCAB_PAD = 128
_G = 128


def _sc_gather(W_pe, pe_flat, W_h, h_flat):
    n_idx = pe_flat.shape[0]
    d = W_pe.shape[1]
    info = plsc.get_sparse_core_info()
    nw = info.num_cores * info.num_subcores
    per_w = n_idx // nw
    mesh = plsc.VectorSubcoreMesh(core_axis_name="c", subcore_axis_name="s")

    @functools.partial(
        pl.kernel,
        mesh=mesh,
        compiler_params=pltpu.CompilerParams(use_tc_tiling_on_sc=False),
        out_type=[
            jax.ShapeDtypeStruct((n_idx, d), jnp.float32),
            jax.ShapeDtypeStruct((n_idx, d), jnp.float32),
        ],
        scratch_types=[
            pltpu.VMEM((per_w,), jnp.int32),
            pltpu.VMEM((per_w, d), jnp.float32),
            pltpu.VMEM((per_w,), jnp.int32),
            pltpu.VMEM((per_w, d), jnp.float32),
            pltpu.SemaphoreType.DMA,
            pltpu.SemaphoreType.DMA,
        ],
    )
    def k(wpe_hbm, pe_hbm, wh_hbm, h_hbm, pe_out, h_out,
          idx_pe, rows_pe, idx_h, rows_h, sem_pe, sem_h):
        wid = lax.axis_index("s") * info.num_cores + lax.axis_index("c")
        base = wid * per_w
        pltpu.sync_copy(pe_hbm.at[pl.ds(base, per_w)], idx_pe)
        pltpu.sync_copy(h_hbm.at[pl.ds(base, per_w)], idx_h)
        cp_pe = pltpu.async_copy(wpe_hbm.at[idx_pe], rows_pe, sem_pe)
        cp_h = pltpu.async_copy(wh_hbm.at[idx_h], rows_h, sem_h)
        cp_pe.wait()
        cp_h.wait()
        pltpu.sync_copy(rows_pe, pe_out.at[pl.ds(base, per_w)])
        pltpu.sync_copy(rows_h, h_out.at[pl.ds(base, per_w)])

    return k(W_pe, pe_flat, W_h, h_flat)


def _tc_body(e_ref, et_ref, pe_ref, hrow_ref, wet_ref, eout_ref, hout_ref):
    pe = pe_ref[0]
    pet = pe.T
    hout_ref[0] = hrow_ref[0].T + pet
    g = pl.program_id(1)
    base = g * _G
    eb_rows = e_ref[0, pl.ds(base, _G), :]
    ebt_rows = et_ref[0, pl.ds(base, _G), :]
    row_ids = lax.broadcasted_iota(jnp.int32, (_G, _N), 0) + base
    col_ids = lax.broadcasted_iota(jnp.int32, (_G, _N), 1)
    esym_g = jnp.where(row_ids > col_ids, ebt_rows, eb_rows)
    petg = pe_ref[0, pl.ds(base, _G), :].T
    wet = wet_ref[...].astype(jnp.bfloat16)
    vocab_col = lax.broadcasted_iota(jnp.int32, (_EVOCAB_PAD, 1), 0)
    for k in range(_G // 2):
        ids2 = jnp.concatenate(
            [esym_g[2 * k:2 * k + 1, :], esym_g[2 * k + 1:2 * k + 2, :]],
            axis=1)
        oht = (vocab_col == ids2).astype(jnp.bfloat16)
        res = lax.dot_general(wet, oht, (((1,), (0,)), ((), ())),
                              preferred_element_type=jnp.float32)
        eout_ref[0, 2 * k] = (res[:, :_N] + pet) + petg[:, 2 * k:2 * k + 1]
        eout_ref[0, 2 * k + 1] = (
            res[:, _N:] + pet) + petg[:, 2 * k + 1:2 * k + 2]


def _tc_call(e, e_t, pe_emb, h_rows, wet_pad):
    return pl.pallas_call(
        _tc_body,
        grid=(_B, _N // _G),
        compiler_params=pltpu.CompilerParams(
            dimension_semantics=("arbitrary", "arbitrary"),
            vmem_limit_bytes=100 * 1024 * 1024,
        ),
        in_specs=[
            pl.BlockSpec((1, _N, _N), lambda b, g: (b, 0, 0)),
            pl.BlockSpec((1, _N, _N), lambda b, g: (b, 0, 0)),
            pl.BlockSpec((1, _N, _D), lambda b, g: (b, 0, 0)),
            pl.BlockSpec((1, _N, _D), lambda b, g: (b, 0, 0)),
            pl.BlockSpec((_D, _EVOCAB_PAD), lambda b, g: (0, 0)),
        ],
        out_specs=[
            pl.BlockSpec((1, _G, _D, _N), lambda b, g: (b, g, 0, 0)),
            pl.BlockSpec((1, _D, _N), lambda b, g: (b, 0, 0)),
        ],
        out_shape=[
            jax.ShapeDtypeStruct((_B, _N, _D, _N), jnp.float32),
            jax.ShapeDtypeStruct((_B, _D, _N), jnp.float32),
        ],
    )(e, e_t, pe_emb, h_rows, wet_pad)


@jax.jit
def kernel(h, e, pe, W_h, W_e, W_pe):
    h = h.astype(jnp.int32)
    e = e.astype(jnp.int32)
    pe = pe.astype(jnp.int32)
    pe_emb_flat, h_rows_flat = _sc_gather(
        W_pe, pe.reshape(_B * _N), W_h, h.reshape(_B * _N))
    pe_emb = pe_emb_flat.reshape(_B, _N, _D)
    h_rows = h_rows_flat.reshape(_B, _N, _D)
    wet_pad = jnp.zeros((_D, _EVOCAB_PAD), jnp.float32).at[:, :W_e.shape[0]].set(W_e.T)
    e_t = jnp.swapaxes(e, 1, 2)
    e_out_t, h_out_t = _tc_call(e, e_t, pe_emb, h_rows, wet_pad)
    e_out = jnp.transpose(e_out_t, (0, 1, 3, 2))
    h_out = jnp.transpose(h_out_t, (0, 2, 1))
    return (h_out, e_out)


# speedup vs baseline: 1.6707x; 1.0637x over previous
import functools

import jax
import jax.numpy as jnp
from jax import lax
from jax.experimental import pallas as pl
from jax.experimental.pallas import tpu as pltpu
from jax.experimental.pallas import tpu_sc as plsc

_B, _N, _D = 64, 128, 64
_EVOCAB_PAD = 128
_BB = 2


def _sc_gather(W_pe, pe_flat, W_h, h_flat):
    n_idx = pe_flat.shape[0]
    d = W_pe.shape[1]
    info = plsc.get_sparse_core_info()
    nw = info.num_cores * info.num_subcores
    per_w = n_idx // nw
    mesh = plsc.VectorSubcoreMesh(core_axis_name="c", subcore_axis_name="s")

    @functools.partial(
        pl.kernel,
        mesh=mesh,
        compiler_params=pltpu.CompilerParams(use_tc_tiling_on_sc=False),
        out_type=[
            jax.ShapeDtypeStruct((n_idx, d), jnp.float32),
            jax.ShapeDtypeStruct((n_idx, d), jnp.float32),
        ],
        scratch_types=[
            pltpu.VMEM((per_w,), jnp.int32),
            pltpu.VMEM((per_w, d), jnp.float32),
            pltpu.VMEM((per_w,), jnp.int32),
            pltpu.VMEM((per_w, d), jnp.float32),
            pltpu.SemaphoreType.DMA,
            pltpu.SemaphoreType.DMA,
        ],
    )
    def k(wpe_hbm, pe_hbm, wh_hbm, h_hbm, pe_out, h_out,
          idx_pe, rows_pe, idx_h, rows_h, sem_pe, sem_h):
        wid = lax.axis_index("s") * info.num_cores + lax.axis_index("c")
        base = wid * per_w
        pltpu.sync_copy(pe_hbm.at[pl.ds(base, per_w)], idx_pe)
        pltpu.sync_copy(h_hbm.at[pl.ds(base, per_w)], idx_h)
        cp_pe = pltpu.async_copy(wpe_hbm.at[idx_pe], rows_pe, sem_pe)
        cp_h = pltpu.async_copy(wh_hbm.at[idx_h], rows_h, sem_h)
        cp_pe.wait()
        cp_h.wait()
        pltpu.sync_copy(rows_pe, pe_out.at[pl.ds(base, per_w)])
        pltpu.sync_copy(rows_h, h_out.at[pl.ds(base, per_w)])

    return k(W_pe, pe_flat, W_h, h_flat)


def _tc_body(e_ref, et_ref, pe_ref, hrow_ref, wet_ref, eout_ref, hout_ref):
    wet = wet_ref[...].astype(jnp.bfloat16)
    vocab_col = lax.broadcasted_iota(jnp.int32, (_EVOCAB_PAD, 1), 0)
    row_ids = lax.broadcasted_iota(jnp.int32, (_N, _N), 0)
    col_ids = lax.broadcasted_iota(jnp.int32, (_N, _N), 1)
    for bb in range(_BB):
        pe = pe_ref[bb]
        pet = pe.T
        hout_ref[bb] = hrow_ref[bb].T + pet
        esym = jnp.where(row_ids > col_ids, et_ref[bb], e_ref[bb])
        for k in range(_N // 2):
            ids2 = jnp.concatenate(
                [esym[2 * k:2 * k + 1, :], esym[2 * k + 1:2 * k + 2, :]],
                axis=1)
            oht = (vocab_col == ids2).astype(jnp.bfloat16)
            res = lax.dot_general(wet, oht, (((1,), (0,)), ((), ())),
                                  preferred_element_type=jnp.float32)
            eout_ref[bb, 2 * k] = (res[:, :_N] + pet) + pet[:, 2 * k:2 * k + 1]
            eout_ref[bb, 2 * k + 1] = (
                res[:, _N:] + pet) + pet[:, 2 * k + 1:2 * k + 2]


def _tc_call(e, e_t, pe_emb, h_rows, wet_pad):
    return pl.pallas_call(
        _tc_body,
        grid=(_B // _BB,),
        compiler_params=pltpu.CompilerParams(
            dimension_semantics=("arbitrary",),
            vmem_limit_bytes=100 * 1024 * 1024,
        ),
        in_specs=[
            pl.BlockSpec((_BB, _N, _N), lambda b: (b, 0, 0)),
            pl.BlockSpec((_BB, _N, _N), lambda b: (b, 0, 0)),
            pl.BlockSpec((_BB, _N, _D), lambda b: (b, 0, 0)),
            pl.BlockSpec((_BB, _N, _D), lambda b: (b, 0, 0)),
            pl.BlockSpec((_D, _EVOCAB_PAD), lambda b: (0, 0)),
        ],
        out_specs=[
            pl.BlockSpec((_BB, _N, _D, _N), lambda b: (b, 0, 0, 0)),
            pl.BlockSpec((_BB, _D, _N), lambda b: (b, 0, 0)),
        ],
        out_shape=[
            jax.ShapeDtypeStruct((_B, _N, _D, _N), jnp.float32),
            jax.ShapeDtypeStruct((_B, _D, _N), jnp.float32),
        ],
    )(e, e_t, pe_emb, h_rows, wet_pad)


@jax.jit
def kernel(h, e, pe, W_h, W_e, W_pe):
    h = h.astype(jnp.int32)
    e = e.astype(jnp.int32)
    pe = pe.astype(jnp.int32)
    pe_emb_flat, h_rows_flat = _sc_gather(
        W_pe, pe.reshape(_B * _N), W_h, h.reshape(_B * _N))
    pe_emb = pe_emb_flat.reshape(_B, _N, _D)
    h_rows = h_rows_flat.reshape(_B, _N, _D)
    wet_pad = jnp.zeros((_D, _EVOCAB_PAD), jnp.float32).at[:, :W_e.shape[0]].set(W_e.T)
    e_t = jnp.swapaxes(e, 1, 2)
    e_out_t, h_out_t = _tc_call(e, e_t, pe_emb, h_rows, wet_pad)
    e_out = jnp.transpose(e_out_t, (0, 1, 3, 2))
    h_out = jnp.transpose(h_out_t, (0, 2, 1))
    return (h_out, e_out)
